# baseline (speedup 1.0000x reference)
import functools

import jax
import jax.numpy as jnp
from jax import lax
from jax.experimental import pallas as pl
from jax.experimental.pallas import tpu as pltpu
from jax.experimental.pallas import tpu_sc as plsc

HID = 128
BS = 1024
N_NUM = 100
N_BIN = 26
CAT_LEN = 50
NAME_LEN = 8
EPS = 1e-5

HDR = 1024
HDR_PAD = 6400
B_TILE = 128
HALF = BS // 2
HALF_CAT = HALF * CAT_LEN
GRID_H = HALF // B_TILE
N_SEQ = N_NUM + N_BIN + CAT_LEN
N_NB = N_NUM + N_BIN


def _make_sc_gather(hdr_rows, cat_off, cat_base, cat_rows, tot_rows):
    info = plsc.get_sparse_core_info()
    nc, ns = info.num_cores, info.num_subcores
    nw = nc * ns
    hdr_w = hdr_rows // nw
    cat_w = cat_rows // nw
    nbuf = 5

    mesh = plsc.VectorSubcoreMesh(core_axis_name="c", subcore_axis_name="s")

    @functools.partial(
        pl.kernel,
        mesh=mesh,
        out_type=jax.ShapeDtypeStruct((tot_rows, HID), jnp.float32),
        scratch_types=[
            pltpu.VMEM((hdr_w + cat_w,), jnp.int32),
        ] + [pltpu.VMEM((128, HID), jnp.float32)] * nbuf
          + [pltpu.SemaphoreType.DMA] * (2 * nbuf),
    )
    def gather_kernel(table_hbm, *io):
        if hdr_rows:
            hdr_idx_hbm, cat_idx_hbm, out_hbm, idx_v, *rest = io
        else:
            cat_idx_hbm, out_hbm, idx_v, *rest = io
        bufs = rest[:nbuf]
        gsems = rest[nbuf:2 * nbuf]
        wsems = rest[2 * nbuf:3 * nbuf]
        wid = lax.axis_index("s") * nc + lax.axis_index("c")
        if hdr_w:
            pltpu.sync_copy(hdr_idx_hbm.at[pl.ds(wid * hdr_w, hdr_w)],
                            idx_v.at[pl.ds(0, hdr_w)])
        pltpu.sync_copy(cat_idx_hbm.at[pl.ds(cat_off + wid * cat_w, cat_w)],
                        idx_v.at[pl.ds(hdr_w, cat_w)])
        jobs = []
        if hdr_w:
            jobs.append((0, wid * hdr_w, hdr_w))
        off = 0
        while off < cat_w:
            sz = min(128, cat_w - off)
            jobs.append((hdr_w + off, cat_base + wid * cat_w + off, sz))
            off += sz
        gcp = [None] * nbuf
        wcp = [None] * nbuf
        for i, (ioff, obase, sz) in enumerate(jobs):
            b = i % nbuf
            if wcp[b] is not None:
                wcp[b].wait()
            gcp[b] = pltpu.async_copy(
                table_hbm.at[idx_v.at[pl.ds(ioff, sz)]],
                bufs[b].at[pl.ds(0, sz)], gsems[b])
            if i >= 1:
                pb = (i - 1) % nbuf
                pioff, pobase, psz = jobs[i - 1]
                gcp[pb].wait()
                wcp[pb] = pltpu.async_copy(
                    bufs[pb].at[pl.ds(0, psz)],
                    out_hbm.at[pl.ds(pobase, psz)], wsems[pb])
        lb = (len(jobs) - 1) % nbuf
        _, lobase, lsz = jobs[-1]
        gcp[lb].wait()
        wcp[lb] = pltpu.async_copy(
            bufs[lb].at[pl.ds(0, lsz)],
            out_hbm.at[pl.ds(lobase, lsz)], wsems[lb])
        for b in range(nbuf):
            if wcp[b] is not None:
                wcp[b].wait()

    return gather_kernel


def _fold_pq(hdr, nm, bm, g, lb, nbias, bbias, wn, bn, wb, bbin):
    mu = jnp.mean(hdr, axis=-1, keepdims=True)
    var = jnp.mean((hdr - mu) ** 2, axis=-1, keepdims=True)
    ln_h = (hdr - mu) * lax.rsqrt(var + EPS) * g + lb
    num_col = ((ln_h[:N_NUM * NAME_LEN].reshape(N_NUM, NAME_LEN, HID)
                * nm[:, :, None]).sum(axis=1)
               / nm.sum(axis=1, keepdims=True))
    bin_base = N_NUM * NAME_LEN
    bin_col = ((ln_h[bin_base:bin_base + N_BIN * NAME_LEN]
                .reshape(N_BIN, NAME_LEN, HID) * bm[:, :, None]).sum(axis=1)
               / bm.sum(axis=1, keepdims=True))
    p_num = jnp.dot(num_col, wn, preferred_element_type=jnp.float32)
    q_num = jnp.dot(nbias, wn, preferred_element_type=jnp.float32) + bn
    p_bin = jnp.dot(bin_col, wb, preferred_element_type=jnp.float32)
    q_bin = jnp.dot(bbias, wb, preferred_element_type=jnp.float32) + bbin
    p = jnp.concatenate([p_num, p_bin], axis=0)
    q = jnp.concatenate([jnp.broadcast_to(q_num, (N_NUM, HID)),
                         jnp.broadcast_to(q_bin, (N_BIN, HID))], axis=0)
    return p, q


def _dense_common(p, q, cat_ref, xn_ref, xb_ref, cm_ref, g, lb,
                  wc_ref, bc_ref, emb_ref, mask_ref):
    x = jnp.concatenate([xn_ref[...], xb_ref[...]], axis=1)
    numbin = x[:, :, None] * p[None] + q[None]
    cat = cat_ref[...]
    cmu = jnp.mean(cat, axis=-1, keepdims=True)
    cvar = jnp.mean((cat - cmu) ** 2, axis=-1, keepdims=True)
    cln = (cat - cmu) * lax.rsqrt(cvar + EPS) * g + lb
    catf = jnp.dot(cln, wc_ref[...], preferred_element_type=jnp.float32) + bc_ref[...]
    emb_ref[...] = jnp.concatenate(
        [numbin, catf.reshape(B_TILE, CAT_LEN, HID)], axis=1)
    mask_ref[...] = jnp.concatenate(
        [jnp.ones((B_TILE, N_NB), jnp.float32), cm_ref[...]], axis=1)


def _dense1_body(hdr_ref, cat_ref, xn_ref, xb_ref, nm_ref, bm_ref, cm_ref,
                 g_ref, lb_ref, nbias_ref, bbias_ref,
                 wn_ref, bn_ref, wc_ref, bc_ref, wb_ref, bbin_ref,
                 emb_ref, mask_ref, p_out, q_out, p_scr, q_scr):
    g = g_ref[...]
    lb = lb_ref[...]

    @pl.when(pl.program_id(0) == 0)
    def _():
        p, q = _fold_pq(hdr_ref[...], nm_ref[...], bm_ref[...], g, lb,
                        nbias_ref[...], bbias_ref[...],
                        wn_ref[...], bn_ref[...], wb_ref[...], bbin_ref[...])
        p_scr[...] = p
        q_scr[...] = q
        p_out[...] = p
        q_out[...] = q

    _dense_common(p_scr[...], q_scr[...], cat_ref, xn_ref, xb_ref, cm_ref,
                  g, lb, wc_ref, bc_ref, emb_ref, mask_ref)


def _dense2_body(p_ref, q_ref, cat_ref, xn_ref, xb_ref, cm_ref,
                 g_ref, lb_ref, wc_ref, bc_ref,
                 emb_in_ref, mask_in_ref, emb_ref, mask_ref):
    del emb_in_ref, mask_in_ref
    _dense_common(p_ref[...], q_ref[...], cat_ref, xn_ref, xb_ref, cm_ref,
                  g_ref[...], lb_ref[...], wc_ref, bc_ref, emb_ref, mask_ref)


def _const2(shape):
    return pl.BlockSpec(shape, lambda i: (0, 0))


_OUT_SHAPE_MAIN = [
    jax.ShapeDtypeStruct((BS, N_SEQ, HID), jnp.float32),
    jax.ShapeDtypeStruct((BS, N_SEQ), jnp.float32),
]


def _dense_half1(rows1, x_num, x_bin, num_mask, bin_mask, cat_mask,
                 ln_g, ln_b, num_bias, bin_bias,
                 w_num, b_num, w_cat, b_cat, w_bin, b_bin):
    return pl.pallas_call(
        _dense1_body,
        grid=(GRID_H,),
        in_specs=[
            pl.BlockSpec((HDR, HID), lambda i: (0, 0)),
            pl.BlockSpec((B_TILE * CAT_LEN, HID), lambda i: (i + 1, 0)),
            pl.BlockSpec((B_TILE, N_NUM), lambda i: (i, 0)),
            pl.BlockSpec((B_TILE, N_BIN), lambda i: (i, 0)),
            _const2((N_NUM, NAME_LEN)),
            _const2((N_BIN, NAME_LEN)),
            pl.BlockSpec((B_TILE, CAT_LEN), lambda i: (i, 0)),
            _const2((1, HID)), _const2((1, HID)),
            _const2((1, HID)), _const2((1, HID)),
            _const2((HID, HID)), _const2((1, HID)),
            _const2((HID, HID)), _const2((1, HID)),
            _const2((HID, HID)), _const2((1, HID)),
        ],
        out_specs=[
            pl.BlockSpec((B_TILE, N_SEQ, HID), lambda i: (i, 0, 0)),
            pl.BlockSpec((B_TILE, N_SEQ), lambda i: (i, 0)),
            _const2((N_NB, HID)),
            _const2((N_NB, HID)),
        ],
        out_shape=_OUT_SHAPE_MAIN + [
            jax.ShapeDtypeStruct((N_NB, HID), jnp.float32),
            jax.ShapeDtypeStruct((N_NB, HID), jnp.float32),
        ],
        scratch_shapes=[
            pltpu.VMEM((N_NB, HID), jnp.float32),
            pltpu.VMEM((N_NB, HID), jnp.float32),
        ],
    )(rows1, rows1, x_num, x_bin, num_mask, bin_mask, cat_mask,
      ln_g, ln_b, num_bias, bin_bias,
      w_num, b_num, w_cat, b_cat, w_bin, b_bin)


def _dense_half2(p, q, rows2, x_num, x_bin, cat_mask, ln_g, ln_b,
                 w_cat, b_cat, emb_in, mask_in):
    t0 = GRID_H
    return pl.pallas_call(
        _dense2_body,
        grid=(GRID_H,),
        in_specs=[
            _const2((N_NB, HID)),
            _const2((N_NB, HID)),
            pl.BlockSpec((B_TILE * CAT_LEN, HID), lambda i: (i, 0)),
            pl.BlockSpec((B_TILE, N_NUM), lambda i: (i + t0, 0)),
            pl.BlockSpec((B_TILE, N_BIN), lambda i: (i + t0, 0)),
            pl.BlockSpec((B_TILE, CAT_LEN), lambda i: (i + t0, 0)),
            _const2((1, HID)), _const2((1, HID)),
            _const2((HID, HID)), _const2((1, HID)),
            pl.BlockSpec(memory_space=pltpu.MemorySpace.HBM),
            pl.BlockSpec(memory_space=pltpu.MemorySpace.HBM),
        ],
        out_specs=[
            pl.BlockSpec((B_TILE, N_SEQ, HID), lambda i: (i + t0, 0, 0)),
            pl.BlockSpec((B_TILE, N_SEQ), lambda i: (i + t0, 0)),
        ],
        out_shape=_OUT_SHAPE_MAIN,
        input_output_aliases={10: 0, 11: 1},
    )(p, q, rows2, x_num, x_bin, cat_mask, ln_g, ln_b, w_cat, b_cat,
      emb_in, mask_in)


def kernel(x_num, num_col_input_ids, num_att_mask, x_cat_input_ids,
           cat_att_mask, x_bin, x_bin_input_ids, bin_att_mask, word_table,
           ln_g, ln_b, num_bias, bin_bias, W_num, b_num, W_cat, b_cat,
           W_bin, b_bin):
    cat_ids = x_cat_input_ids.reshape(-1).astype(jnp.int32)
    hdr_ids = jnp.concatenate([
        num_col_input_ids.reshape(-1).astype(jnp.int32),
        x_bin_input_ids.reshape(-1).astype(jnp.int32),
        jnp.zeros((HDR - (N_NUM + N_BIN) * NAME_LEN,), jnp.int32),
    ])
    gather1 = _make_sc_gather(HDR, 0, HDR_PAD, HALF_CAT, HDR_PAD + HALF_CAT)
    gather2 = _make_sc_gather(0, HALF_CAT, 0, HALF_CAT, HALF_CAT)
    rows1 = gather1(word_table, hdr_ids, cat_ids)
    rows2 = gather2(word_table, cat_ids)

    nm = num_att_mask.astype(jnp.float32)
    bm = bin_att_mask.astype(jnp.float32)
    cm = cat_att_mask.astype(jnp.float32)
    emb1, mask1, p, q = _dense_half1(
        rows1, x_num, x_bin, nm, bm, cm,
        ln_g.reshape(1, HID), ln_b.reshape(1, HID),
        num_bias.reshape(1, HID), bin_bias.reshape(1, HID),
        W_num, b_num.reshape(1, HID), W_cat, b_cat.reshape(1, HID),
        W_bin, b_bin.reshape(1, HID))
    emb, mask = _dense_half2(
        p, q, rows2, x_num, x_bin, cm,
        ln_g.reshape(1, HID), ln_b.reshape(1, HID),
        W_cat, b_cat.reshape(1, HID), emb1, mask1)
    return emb, mask


# speedup vs baseline: 1.0244x; 1.0244x over previous
import functools

import jax
import jax.numpy as jnp
from jax import lax
from jax.experimental import pallas as pl
from jax.experimental.pallas import tpu as pltpu
from jax.experimental.pallas import tpu_sc as plsc

HID = 128
BS = 1024
N_NUM = 100
N_BIN = 26
CAT_LEN = 50
NAME_LEN = 8
EPS = 1e-5

HDR = 1024
HDR_PAD = 6400
B_TILE = 128
TILES = BS // B_TILE
T1 = 2
T2 = TILES - T1
CAT1 = T1 * B_TILE * CAT_LEN
CAT2 = T2 * B_TILE * CAT_LEN
N_SEQ = N_NUM + N_BIN + CAT_LEN
N_NB = N_NUM + N_BIN


def _make_sc_gather(hdr_rows, cat_off, cat_base, cat_rows, tot_rows):
    info = plsc.get_sparse_core_info()
    nc, ns = info.num_cores, info.num_subcores
    nw = nc * ns
    hdr_w = hdr_rows // nw
    cat_w = cat_rows // nw
    nbuf = 5

    mesh = plsc.VectorSubcoreMesh(core_axis_name="c", subcore_axis_name="s")

    @functools.partial(
        pl.kernel,
        mesh=mesh,
        out_type=jax.ShapeDtypeStruct((tot_rows, HID), jnp.float32),
        scratch_types=[
            pltpu.VMEM((hdr_w + cat_w,), jnp.int32),
        ] + [pltpu.VMEM((128, HID), jnp.float32)] * nbuf
          + [pltpu.SemaphoreType.DMA] * (2 * nbuf),
    )
    def gather_kernel(table_hbm, *io):
        if hdr_rows:
            hdr_idx_hbm, cat_idx_hbm, out_hbm, idx_v, *rest = io
        else:
            cat_idx_hbm, out_hbm, idx_v, *rest = io
        bufs = rest[:nbuf]
        gsems = rest[nbuf:2 * nbuf]
        wsems = rest[2 * nbuf:3 * nbuf]
        wid = lax.axis_index("s") * nc + lax.axis_index("c")
        if hdr_w:
            pltpu.sync_copy(hdr_idx_hbm.at[pl.ds(wid * hdr_w, hdr_w)],
                            idx_v.at[pl.ds(0, hdr_w)])
        pltpu.sync_copy(cat_idx_hbm.at[pl.ds(cat_off + wid * cat_w, cat_w)],
                        idx_v.at[pl.ds(hdr_w, cat_w)])
        jobs = []
        if hdr_w:
            jobs.append((0, wid * hdr_w, hdr_w))
        off = 0
        while off < cat_w:
            sz = min(128, cat_w - off)
            jobs.append((hdr_w + off, cat_base + wid * cat_w + off, sz))
            off += sz
        ahead = nbuf - 2
        n = len(jobs)
        gcp = [None] * n
        wcp = [None] * n
        for i in range(n + ahead):
            if i < n:
                ioff, obase, sz = jobs[i]
                b = i % nbuf
                if i >= nbuf:
                    wcp[i - nbuf].wait()
                gcp[i] = pltpu.async_copy(
                    table_hbm.at[idx_v.at[pl.ds(ioff, sz)]],
                    bufs[b].at[pl.ds(0, sz)], gsems[b])
            j = i - ahead
            if 0 <= j < n:
                _, jobase, jsz = jobs[j]
                jb = j % nbuf
                gcp[j].wait()
                wcp[j] = pltpu.async_copy(
                    bufs[jb].at[pl.ds(0, jsz)],
                    out_hbm.at[pl.ds(jobase, jsz)], wsems[jb])
        for j in range(max(0, n - nbuf), n):
            wcp[j].wait()

    return gather_kernel


def _fold_pq(hdr, nm, bm, g, lb, nbias, bbias, wn, bn, wb, bbin):
    mu = jnp.mean(hdr, axis=-1, keepdims=True)
    var = jnp.mean((hdr - mu) ** 2, axis=-1, keepdims=True)
    ln_h = (hdr - mu) * lax.rsqrt(var + EPS) * g + lb
    num_col = ((ln_h[:N_NUM * NAME_LEN].reshape(N_NUM, NAME_LEN, HID)
                * nm[:, :, None]).sum(axis=1)
               / nm.sum(axis=1, keepdims=True))
    bin_base = N_NUM * NAME_LEN
    bin_col = ((ln_h[bin_base:bin_base + N_BIN * NAME_LEN]
                .reshape(N_BIN, NAME_LEN, HID) * bm[:, :, None]).sum(axis=1)
               / bm.sum(axis=1, keepdims=True))
    p_num = jnp.dot(num_col, wn, preferred_element_type=jnp.float32)
    q_num = jnp.dot(nbias, wn, preferred_element_type=jnp.float32) + bn
    p_bin = jnp.dot(bin_col, wb, preferred_element_type=jnp.float32)
    q_bin = jnp.dot(bbias, wb, preferred_element_type=jnp.float32) + bbin
    p = jnp.concatenate([p_num, p_bin], axis=0)
    q = jnp.concatenate([jnp.broadcast_to(q_num, (N_NUM, HID)),
                         jnp.broadcast_to(q_bin, (N_BIN, HID))], axis=0)
    return p, q


def _dense_common(p, q, cat_ref, xn_ref, xb_ref, cm_ref, g, lb,
                  wc_ref, bc_ref, emb_ref, mask_ref):
    x = jnp.concatenate([xn_ref[...], xb_ref[...]], axis=1)
    numbin = x[:, :, None] * p[None] + q[None]
    cat = cat_ref[...]
    cmu = jnp.mean(cat, axis=-1, keepdims=True)
    cvar = jnp.mean((cat - cmu) ** 2, axis=-1, keepdims=True)
    cln = (cat - cmu) * lax.rsqrt(cvar + EPS) * g + lb
    catf = jnp.dot(cln, wc_ref[...], preferred_element_type=jnp.float32) + bc_ref[...]
    emb_ref[...] = jnp.concatenate(
        [numbin, catf.reshape(B_TILE, CAT_LEN, HID)], axis=1)
    mask_ref[...] = jnp.concatenate(
        [jnp.ones((B_TILE, N_NB), jnp.float32), cm_ref[...]], axis=1)


def _dense1_body(hdr_ref, cat_ref, xn_ref, xb_ref, nm_ref, bm_ref, cm_ref,
                 g_ref, lb_ref, nbias_ref, bbias_ref,
                 wn_ref, bn_ref, wc_ref, bc_ref, wb_ref, bbin_ref,
                 emb_ref, mask_ref, p_out, q_out, p_scr, q_scr):
    g = g_ref[...]
    lb = lb_ref[...]

    @pl.when(pl.program_id(0) == 0)
    def _():
        p, q = _fold_pq(hdr_ref[...], nm_ref[...], bm_ref[...], g, lb,
                        nbias_ref[...], bbias_ref[...],
                        wn_ref[...], bn_ref[...], wb_ref[...], bbin_ref[...])
        p_scr[...] = p
        q_scr[...] = q
        p_out[...] = p
        q_out[...] = q

    _dense_common(p_scr[...], q_scr[...], cat_ref, xn_ref, xb_ref, cm_ref,
                  g, lb, wc_ref, bc_ref, emb_ref, mask_ref)


def _dense2_body(p_ref, q_ref, cat_ref, xn_ref, xb_ref, cm_ref,
                 g_ref, lb_ref, wc_ref, bc_ref,
                 emb_in_ref, mask_in_ref, emb_ref, mask_ref):
    del emb_in_ref, mask_in_ref
    _dense_common(p_ref[...], q_ref[...], cat_ref, xn_ref, xb_ref, cm_ref,
                  g_ref[...], lb_ref[...], wc_ref, bc_ref, emb_ref, mask_ref)


def _const2(shape):
    return pl.BlockSpec(shape, lambda i: (0, 0))


_OUT_SHAPE_MAIN = [
    jax.ShapeDtypeStruct((BS, N_SEQ, HID), jnp.float32),
    jax.ShapeDtypeStruct((BS, N_SEQ), jnp.float32),
]


def _dense_half1(rows1, x_num, x_bin, num_mask, bin_mask, cat_mask,
                 ln_g, ln_b, num_bias, bin_bias,
                 w_num, b_num, w_cat, b_cat, w_bin, b_bin):
    return pl.pallas_call(
        _dense1_body,
        grid=(T1,),
        in_specs=[
            pl.BlockSpec((HDR, HID), lambda i: (0, 0)),
            pl.BlockSpec((B_TILE * CAT_LEN, HID), lambda i: (i + 1, 0)),
            pl.BlockSpec((B_TILE, N_NUM), lambda i: (i, 0)),
            pl.BlockSpec((B_TILE, N_BIN), lambda i: (i, 0)),
            _const2((N_NUM, NAME_LEN)),
            _const2((N_BIN, NAME_LEN)),
            pl.BlockSpec((B_TILE, CAT_LEN), lambda i: (i, 0)),
            _const2((1, HID)), _const2((1, HID)),
            _const2((1, HID)), _const2((1, HID)),
            _const2((HID, HID)), _const2((1, HID)),
            _const2((HID, HID)), _const2((1, HID)),
            _const2((HID, HID)), _const2((1, HID)),
        ],
        out_specs=[
            pl.BlockSpec((B_TILE, N_SEQ, HID), lambda i: (i, 0, 0)),
            pl.BlockSpec((B_TILE, N_SEQ), lambda i: (i, 0)),
            _const2((N_NB, HID)),
            _const2((N_NB, HID)),
        ],
        out_shape=_OUT_SHAPE_MAIN + [
            jax.ShapeDtypeStruct((N_NB, HID), jnp.float32),
            jax.ShapeDtypeStruct((N_NB, HID), jnp.float32),
        ],
        scratch_shapes=[
            pltpu.VMEM((N_NB, HID), jnp.float32),
            pltpu.VMEM((N_NB, HID), jnp.float32),
        ],
    )(rows1, rows1, x_num, x_bin, num_mask, bin_mask, cat_mask,
      ln_g, ln_b, num_bias, bin_bias,
      w_num, b_num, w_cat, b_cat, w_bin, b_bin)


def _dense_half2(p, q, rows2, x_num, x_bin, cat_mask, ln_g, ln_b,
                 w_cat, b_cat, emb_in, mask_in):
    t0 = T1
    return pl.pallas_call(
        _dense2_body,
        grid=(T2,),
        in_specs=[
            _const2((N_NB, HID)),
            _const2((N_NB, HID)),
            pl.BlockSpec((B_TILE * CAT_LEN, HID), lambda i: (i, 0)),
            pl.BlockSpec((B_TILE, N_NUM), lambda i: (i + t0, 0)),
            pl.BlockSpec((B_TILE, N_BIN), lambda i: (i + t0, 0)),
            pl.BlockSpec((B_TILE, CAT_LEN), lambda i: (i + t0, 0)),
            _const2((1, HID)), _const2((1, HID)),
            _const2((HID, HID)), _const2((1, HID)),
            pl.BlockSpec(memory_space=pltpu.MemorySpace.HBM),
            pl.BlockSpec(memory_space=pltpu.MemorySpace.HBM),
        ],
        out_specs=[
            pl.BlockSpec((B_TILE, N_SEQ, HID), lambda i: (i + t0, 0, 0)),
            pl.BlockSpec((B_TILE, N_SEQ), lambda i: (i + t0, 0)),
        ],
        out_shape=_OUT_SHAPE_MAIN,
        input_output_aliases={10: 0, 11: 1},
    )(p, q, rows2, x_num, x_bin, cat_mask, ln_g, ln_b, w_cat, b_cat,
      emb_in, mask_in)


def kernel(x_num, num_col_input_ids, num_att_mask, x_cat_input_ids,
           cat_att_mask, x_bin, x_bin_input_ids, bin_att_mask, word_table,
           ln_g, ln_b, num_bias, bin_bias, W_num, b_num, W_cat, b_cat,
           W_bin, b_bin):
    cat_ids = x_cat_input_ids.reshape(-1).astype(jnp.int32)
    hdr_ids = jnp.concatenate([
        num_col_input_ids.reshape(-1).astype(jnp.int32),
        x_bin_input_ids.reshape(-1).astype(jnp.int32),
        jnp.zeros((HDR - (N_NUM + N_BIN) * NAME_LEN,), jnp.int32),
    ])
    gather1 = _make_sc_gather(HDR, 0, HDR_PAD, CAT1, HDR_PAD + CAT1)
    gather2 = _make_sc_gather(0, CAT1, 0, CAT2, CAT2)
    rows1 = gather1(word_table, hdr_ids, cat_ids)
    rows2 = gather2(word_table, cat_ids)

    nm = num_att_mask.astype(jnp.float32)
    bm = bin_att_mask.astype(jnp.float32)
    cm = cat_att_mask.astype(jnp.float32)
    emb1, mask1, p, q = _dense_half1(
        rows1, x_num, x_bin, nm, bm, cm,
        ln_g.reshape(1, HID), ln_b.reshape(1, HID),
        num_bias.reshape(1, HID), bin_bias.reshape(1, HID),
        W_num, b_num.reshape(1, HID), W_cat, b_cat.reshape(1, HID),
        W_bin, b_bin.reshape(1, HID))
    emb, mask = _dense_half2(
        p, q, rows2, x_num, x_bin, cm,
        ln_g.reshape(1, HID), ln_b.reshape(1, HID),
        W_cat, b_cat.reshape(1, HID), emb1, mask1)
    return emb, mask


# speedup vs baseline: 1.0375x; 1.0128x over previous
import functools

import jax
import jax.numpy as jnp
from jax import lax
from jax.experimental import pallas as pl
from jax.experimental.pallas import tpu as pltpu
from jax.experimental.pallas import tpu_sc as plsc

HID = 128
BS = 1024
N_NUM = 100
N_BIN = 26
CAT_LEN = 50
NAME_LEN = 8
EPS = 1e-5

HDR = 1024
HDR_PAD = 6400
B_TILE = 128
TILES = BS // B_TILE
T1 = 2
T2 = TILES - T1
CAT1 = T1 * B_TILE * CAT_LEN
CAT2 = T2 * B_TILE * CAT_LEN
N_SEQ = N_NUM + N_BIN + CAT_LEN
N_NB = N_NUM + N_BIN


def _make_sc_gather(hdr_rows, cat_off, cat_base, cat_rows, tot_rows):
    info = plsc.get_sparse_core_info()
    nc, ns = info.num_cores, info.num_subcores
    nw = nc * ns
    hdr_w = hdr_rows // nw
    cat_w = cat_rows // nw
    nbuf = 5

    mesh = plsc.VectorSubcoreMesh(core_axis_name="c", subcore_axis_name="s")

    @functools.partial(
        pl.kernel,
        mesh=mesh,
        out_type=jax.ShapeDtypeStruct((tot_rows, HID), jnp.float32),
        scratch_types=[
            pltpu.VMEM((hdr_w + cat_w,), jnp.int32),
        ] + [pltpu.VMEM((128, HID), jnp.float32)] * nbuf
          + [pltpu.SemaphoreType.DMA] * (2 * nbuf),
    )
    def gather_kernel(table_hbm, *io):
        if hdr_rows:
            hdr_idx_hbm, cat_idx_hbm, out_hbm, idx_v, *rest = io
        else:
            cat_idx_hbm, out_hbm, idx_v, *rest = io
        bufs = rest[:nbuf]
        gsems = rest[nbuf:2 * nbuf]
        wsems = rest[2 * nbuf:3 * nbuf]
        wid = lax.axis_index("s") * nc + lax.axis_index("c")
        if hdr_w:
            pltpu.sync_copy(hdr_idx_hbm.at[pl.ds(wid * hdr_w, hdr_w)],
                            idx_v.at[pl.ds(0, hdr_w)])
        pltpu.sync_copy(cat_idx_hbm.at[pl.ds(cat_off + wid * cat_w, cat_w)],
                        idx_v.at[pl.ds(hdr_w, cat_w)])
        jobs = []
        if hdr_w:
            jobs.append((0, wid * hdr_w, hdr_w))
        off = 0
        while off < cat_w:
            sz = min(128, cat_w - off)
            jobs.append((hdr_w + off, cat_base + wid * cat_w + off, sz))
            off += sz
        ahead = nbuf - 2
        n = len(jobs)
        gcp = [None] * n
        wcp = [None] * n
        for i in range(n + ahead):
            if i < n:
                ioff, obase, sz = jobs[i]
                b = i % nbuf
                if i >= nbuf:
                    wcp[i - nbuf].wait()
                gcp[i] = pltpu.async_copy(
                    table_hbm.at[idx_v.at[pl.ds(ioff, sz)]],
                    bufs[b].at[pl.ds(0, sz)], gsems[b])
            j = i - ahead
            if 0 <= j < n:
                _, jobase, jsz = jobs[j]
                jb = j % nbuf
                gcp[j].wait()
                wcp[j] = pltpu.async_copy(
                    bufs[jb].at[pl.ds(0, jsz)],
                    out_hbm.at[pl.ds(jobase, jsz)], wsems[jb])
        for j in range(max(0, n - nbuf), n):
            wcp[j].wait()

    return gather_kernel


def _fold_pq(hdr, nm, bm, g, lb, nbias, bbias, wn, bn, wb, bbin):
    mu = jnp.mean(hdr, axis=-1, keepdims=True)
    var = jnp.mean((hdr - mu) ** 2, axis=-1, keepdims=True)
    ln_h = (hdr - mu) * lax.rsqrt(var + EPS) * g + lb
    num_col = ((ln_h[:N_NUM * NAME_LEN].reshape(N_NUM, NAME_LEN, HID)
                * nm[:, :, None]).sum(axis=1)
               / nm.sum(axis=1, keepdims=True))
    bin_base = N_NUM * NAME_LEN
    bin_col = ((ln_h[bin_base:bin_base + N_BIN * NAME_LEN]
                .reshape(N_BIN, NAME_LEN, HID) * bm[:, :, None]).sum(axis=1)
               / bm.sum(axis=1, keepdims=True))
    p_num = jnp.dot(num_col, wn, preferred_element_type=jnp.float32)
    q_num = jnp.dot(nbias, wn, preferred_element_type=jnp.float32) + bn
    p_bin = jnp.dot(bin_col, wb, preferred_element_type=jnp.float32)
    q_bin = jnp.dot(bbias, wb, preferred_element_type=jnp.float32) + bbin
    p = jnp.concatenate([p_num, p_bin], axis=0)
    q = jnp.concatenate([jnp.broadcast_to(q_num, (N_NUM, HID)),
                         jnp.broadcast_to(q_bin, (N_BIN, HID))], axis=0)
    return p, q


def _dense_common(p, q, cat_ref, xn_ref, xb_ref, cm_ref, g, lb,
                  wc_ref, bc_ref, emb_ref, mask_ref):
    x = jnp.concatenate([xn_ref[...], xb_ref[...]], axis=1)
    numbin = x[:, :, None] * p[None] + q[None]
    cat = cat_ref[...]
    cmu = jnp.mean(cat, axis=-1, keepdims=True)
    cvar = jnp.mean((cat - cmu) ** 2, axis=-1, keepdims=True)
    cln = (cat - cmu) * lax.rsqrt(cvar + EPS) * g + lb
    catf = (jnp.dot(cln, wc_ref[...], preferred_element_type=jnp.float32)
            + bc_ref[...][None, :])
    emb_ref[...] = jnp.concatenate(
        [numbin, catf.reshape(B_TILE, CAT_LEN, HID)], axis=1)
    mask_ref[...] = jnp.concatenate(
        [jnp.ones((B_TILE, N_NB), jnp.float32), cm_ref[...]], axis=1)


def _dense1_body(hdr_ref, cat_ref, xn_ref, xb_ref, nm_ref, bm_ref, cm_ref,
                 g_ref, lb_ref, nbias_ref, bbias_ref,
                 wn_ref, bn_ref, wc_ref, bc_ref, wb_ref, bbin_ref,
                 emb_ref, mask_ref, p_out, q_out, p_scr, q_scr):
    g = g_ref[...][None, :]
    lb = lb_ref[...][None, :]

    @pl.when(pl.program_id(0) == 0)
    def _():
        p, q = _fold_pq(hdr_ref[...], nm_ref[...], bm_ref[...], g, lb,
                        nbias_ref[...].reshape(1, HID),
                        bbias_ref[...].reshape(1, HID),
                        wn_ref[...], bn_ref[...][None, :],
                        wb_ref[...], bbin_ref[...][None, :])
        p_scr[...] = p
        q_scr[...] = q
        p_out[...] = p
        q_out[...] = q

    _dense_common(p_scr[...], q_scr[...], cat_ref, xn_ref, xb_ref, cm_ref,
                  g, lb, wc_ref, bc_ref, emb_ref, mask_ref)


def _dense2_body(p_ref, q_ref, cat_ref, xn_ref, xb_ref, cm_ref,
                 g_ref, lb_ref, wc_ref, bc_ref,
                 emb_in_ref, mask_in_ref, emb_ref, mask_ref):
    del emb_in_ref, mask_in_ref
    _dense_common(p_ref[...], q_ref[...], cat_ref, xn_ref, xb_ref, cm_ref,
                  g_ref[...][None, :], lb_ref[...][None, :],
                  wc_ref, bc_ref, emb_ref, mask_ref)


_vec1 = pl.BlockSpec((HID,), lambda i: (0,))
_b3 = pl.BlockSpec((1, 1, HID), lambda i: (0, 0, 0))


def _const2(shape):
    return pl.BlockSpec(shape, lambda i: (0, 0))


_OUT_SHAPE_MAIN = [
    jax.ShapeDtypeStruct((BS, N_SEQ, HID), jnp.float32),
    jax.ShapeDtypeStruct((BS, N_SEQ), jnp.float32),
]


def _dense_half1(rows1, x_num, x_bin, num_mask, bin_mask, cat_mask,
                 ln_g, ln_b, num_bias, bin_bias,
                 w_num, b_num, w_cat, b_cat, w_bin, b_bin):
    return pl.pallas_call(
        _dense1_body,
        grid=(T1,),
        in_specs=[
            pl.BlockSpec((HDR, HID), lambda i: (0, 0)),
            pl.BlockSpec((B_TILE * CAT_LEN, HID), lambda i: (i + 1, 0)),
            pl.BlockSpec((B_TILE, N_NUM), lambda i: (i, 0)),
            pl.BlockSpec((B_TILE, N_BIN), lambda i: (i, 0)),
            _const2((N_NUM, NAME_LEN)),
            _const2((N_BIN, NAME_LEN)),
            pl.BlockSpec((B_TILE, CAT_LEN), lambda i: (i, 0)),
            _vec1, _vec1,
            _b3, _b3,
            _const2((HID, HID)), _vec1,
            _const2((HID, HID)), _vec1,
            _const2((HID, HID)), _vec1,
        ],
        out_specs=[
            pl.BlockSpec((B_TILE, N_SEQ, HID), lambda i: (i, 0, 0)),
            pl.BlockSpec((B_TILE, N_SEQ), lambda i: (i, 0)),
            _const2((N_NB, HID)),
            _const2((N_NB, HID)),
        ],
        out_shape=_OUT_SHAPE_MAIN + [
            jax.ShapeDtypeStruct((N_NB, HID), jnp.float32),
            jax.ShapeDtypeStruct((N_NB, HID), jnp.float32),
        ],
        scratch_shapes=[
            pltpu.VMEM((N_NB, HID), jnp.float32),
            pltpu.VMEM((N_NB, HID), jnp.float32),
        ],
    )(rows1, rows1, x_num, x_bin, num_mask, bin_mask, cat_mask,
      ln_g, ln_b, num_bias, bin_bias,
      w_num, b_num, w_cat, b_cat, w_bin, b_bin)


def _dense_half2(p, q, rows2, x_num, x_bin, cat_mask, ln_g, ln_b,
                 w_cat, b_cat, emb_in, mask_in):
    t0 = T1
    return pl.pallas_call(
        _dense2_body,
        grid=(T2,),
        in_specs=[
            _const2((N_NB, HID)),
            _const2((N_NB, HID)),
            pl.BlockSpec((B_TILE * CAT_LEN, HID), lambda i: (i, 0)),
            pl.BlockSpec((B_TILE, N_NUM), lambda i: (i + t0, 0)),
            pl.BlockSpec((B_TILE, N_BIN), lambda i: (i + t0, 0)),
            pl.BlockSpec((B_TILE, CAT_LEN), lambda i: (i + t0, 0)),
            _vec1, _vec1,
            _const2((HID, HID)), _vec1,
            pl.BlockSpec(memory_space=pltpu.MemorySpace.HBM),
            pl.BlockSpec(memory_space=pltpu.MemorySpace.HBM),
        ],
        out_specs=[
            pl.BlockSpec((B_TILE, N_SEQ, HID), lambda i: (i + t0, 0, 0)),
            pl.BlockSpec((B_TILE, N_SEQ), lambda i: (i + t0, 0)),
        ],
        out_shape=_OUT_SHAPE_MAIN,
        input_output_aliases={10: 0, 11: 1},
    )(p, q, rows2, x_num, x_bin, cat_mask, ln_g, ln_b, w_cat, b_cat,
      emb_in, mask_in)


def kernel(x_num, num_col_input_ids, num_att_mask, x_cat_input_ids,
           cat_att_mask, x_bin, x_bin_input_ids, bin_att_mask, word_table,
           ln_g, ln_b, num_bias, bin_bias, W_num, b_num, W_cat, b_cat,
           W_bin, b_bin):
    cat_ids = x_cat_input_ids.reshape(-1).astype(jnp.int32)
    hdr_ids = jnp.concatenate([
        num_col_input_ids.reshape(-1).astype(jnp.int32),
        x_bin_input_ids.reshape(-1).astype(jnp.int32),
        jnp.zeros((HDR - (N_NUM + N_BIN) * NAME_LEN,), jnp.int32),
    ])
    gather1 = _make_sc_gather(HDR, 0, HDR_PAD, CAT1, HDR_PAD + CAT1)
    gather2 = _make_sc_gather(0, CAT1, 0, CAT2, CAT2)
    rows1 = gather1(word_table, hdr_ids, cat_ids)
    rows2 = gather2(word_table, cat_ids)

    nm = num_att_mask.astype(jnp.float32)
    bm = bin_att_mask.astype(jnp.float32)
    cm = cat_att_mask.astype(jnp.float32)
    emb1, mask1, p, q = _dense_half1(
        rows1, x_num, x_bin, nm, bm, cm,
        ln_g, ln_b, num_bias, bin_bias,
        W_num, b_num, W_cat, b_cat, W_bin, b_bin)
    emb, mask = _dense_half2(
        p, q, rows2, x_num, x_bin, cm,
        ln_g, ln_b, W_cat, b_cat, emb1, mask1)
    return emb, mask
